# baseline (speedup 1.0000x reference)
import jax
import jax.numpy as jnp
from jax import lax
from jax.experimental import pallas as pl
from jax.experimental.pallas import tpu as pltpu

H = 128
T = 50
D = 16
NWEEK = 4
N = 100
NCAT = 5
NPC = 20
K = 10
E_IN = 800 + N
E20 = 100
E_OUT = 20 + NCAT

_F32 = jnp.float32


def _gru_cell(h, gi, gh):
    i_r = gi[:, 0:H]
    i_z = gi[:, H:2 * H]
    i_n = gi[:, 2 * H:3 * H]
    h_r = gh[:, 0:H]
    h_z = gh[:, H:2 * H]
    h_n = gh[:, 2 * H:3 * H]
    r = jax.nn.sigmoid(i_r + h_r)
    z = jax.nn.sigmoid(i_z + h_z)
    n = jnp.tanh(i_n + r * h_n)
    return (1.0 - z) * n + z * h


def _mm(a, b):
    return jnp.dot(a, b, preferred_element_type=_F32)


def _col_to_row(v, n):
    eye = lax.broadcasted_iota(jnp.int32, (n, n), 0) == lax.broadcasted_iota(
        jnp.int32, (n, n), 1)
    return jnp.sum(jnp.where(eye, jnp.broadcast_to(v, (n, n)), 0.0), axis=0,
                   keepdims=True)


def _gat_dense(xp, a_src_row, a_dst_row, bias_row, src_col, dst_col, dst_row,
               n, e):
    es = jnp.sum(xp * a_src_row, axis=1, keepdims=True)
    ed = jnp.sum(xp * a_dst_row, axis=1, keepdims=True)
    ids_e = lax.broadcasted_iota(jnp.int32, (e, n), 1)
    S = (ids_e == src_col).astype(_F32)
    Dm = (ids_e == dst_col).astype(_F32)
    DmTmask = lax.broadcasted_iota(jnp.int32, (n, e), 0) == dst_row
    DmT = DmTmask.astype(_F32)
    alpha = _mm(S, es) + _mm(Dm, ed)
    alpha = jnp.where(alpha >= 0.0, alpha, 0.2 * alpha)
    alpha_b = jnp.broadcast_to(jnp.reshape(alpha, (1, e)), (n, e))
    amax = jnp.max(jnp.where(DmTmask, alpha_b, -1e30), axis=1, keepdims=True)
    ex = jnp.exp(alpha - _mm(Dm, amax))
    den = _mm(DmT, ex)
    coef = ex / (_mm(Dm, den) + 1e-16)
    gathered = _mm(S, xp)
    out = _mm(DmT, coef * gathered)
    return out + bias_row


def _forward_kernel(
    wt_ref, eWih_ref, eWhh_ref, ebih_ref, ebhh_ref, attW_ref, attb_ref,
    gWih_ref, gWhh_ref, gbih_ref, gbhh_ref, waW_ref, wab_ref,
    igW_ref, igas_ref, igad_ref, igb_ref,
    isrc_ref, idstc_ref, idstr_ref,
    s20_ref, d20r_ref, relW_ref, relb_ref, rootW_ref,
    cgW_ref, cgas_ref, cgad_ref, cgb_ref,
    osrc_ref, odstc_ref, odstr_ref,
    fusW_ref, fusb_ref, regW_ref, regb_ref, clsW_ref, clsb_ref,
    reg_ref, cls_ref,
    hs_ref, ws_ref, ys_ref, gi_ref,
):
    WhhTs = [eWhh_ref[i] for i in range(NWEEK)]
    bhhs = [ebhh_ref[i] for i in range(NWEEK)]

    for w in range(NWEEK):
        gi_ref[w] = _mm(wt_ref[w], eWih_ref[w]) + ebih_ref[w]

    def step(t, hcarry):
        off = pl.multiple_of(t * 104, 8)
        new = []
        for w in range(NWEEK):
            gi = gi_ref[w, pl.ds(off, 104)][0:N]
            gh = _mm(hcarry[w], WhhTs[w]) + bhhs[w]
            hn = _gru_cell(hcarry[w], gi, gh)
            hs_ref[w, pl.ds(t, 1)] = jnp.reshape(hn, (1, N, H))
            new.append(hn)
        return tuple(new)

    lax.fori_loop(0, T, step,
                  tuple(jnp.zeros((N, H), _F32) for _ in range(NWEEK)))

    gWihT = gWih_ref[...]
    gWhhT = gWhh_ref[...]
    gbih = gbih_ref[...]
    gbhh = gbhh_ref[...]
    for i in range(NWEEK):
        X3 = hs_ref[i]
        X = jnp.reshape(X3, (T, N * H))
        aw = _mm(attW_ref[i], X) + attb_ref[i]
        m = jnp.max(aw, axis=0, keepdims=True)
        p = jnp.exp(aw - m)
        p = p / jnp.sum(p, axis=0, keepdims=True)
        p3 = jnp.reshape(p, (T, N, H))
        av = jnp.sum(p3 * X3, axis=0)
        ws_ref[i * 104:i * 104 + N, :] = _mm(av, gWihT) + gbih

    def step2(t, h):
        gi = jnp.concatenate(
            [ws_ref[pl.ds(w * 104 + t, 1), :] for w in range(NWEEK)], axis=0)
        gh = _mm(h, gWhhT) + gbhh
        hn = _gru_cell(h, gi, gh)
        for w in range(NWEEK):
            ys_ref[pl.ds(w * 104 + t, 1), :] = hn[w:w + 1, :]
        return hn

    lax.fori_loop(0, N, step2, jnp.zeros((NWEEK, H), _F32))

    ys_w = [ys_ref[w * 104:w * 104 + N, :] for w in range(NWEEK)]
    aw_s = []
    for s in range(NWEEK):
        acc = jnp.zeros((N, H), _F32)
        for t in range(NWEEK):
            acc = acc + waW_ref[s, t] * ys_w[t]
        aw_s.append(acc + wab_ref[s])
    mx = jnp.maximum(jnp.maximum(aw_s[0], aw_s[1]),
                     jnp.maximum(aw_s[2], aw_s[3]))
    e_s = [jnp.exp(a - mx) for a in aw_s]
    den = e_s[0] + e_s[1] + e_s[2] + e_s[3]
    wav = jnp.zeros((N, H), _F32)
    for s in range(NWEEK):
        wav = wav + (e_s[s] / den) * ys_w[s]

    xp_in = _mm(wav, igW_ref[...])
    inner = _gat_dense(xp_in, igas_ref[...], igad_ref[...], igb_ref[...],
                       isrc_ref[...], idstc_ref[...], idstr_ref[...],
                       N, E_IN)

    ids20 = lax.broadcasted_iota(jnp.int32, (E20, NPC), 1)
    S20 = (ids20 == s20_ref[...]).astype(_F32)
    D20T = (lax.broadcasted_iota(jnp.int32, (NPC, E20), 0)
            == d20r_ref[...]).astype(_F32)
    M20 = _mm(D20T, S20)
    relWT = relW_ref[...]
    rootWT = rootW_ref[...]
    io_i = lax.broadcasted_iota(jnp.int32, (NPC, NPC), 0)
    io_j = lax.broadcasted_iota(jnp.int32, (NPC, NPC), 1)
    lt_idx = io_j < io_i
    iop = lax.broadcasted_iota(jnp.int32, (K, NPC), 0)
    cats = []
    for c in range(NCAT):
        x_c = wav[c * NPC:(c + 1) * NPC, :]
        aggr = _mm(M20, x_c)
        score = _mm(aggr, relWT) + _mm(x_c, rootWT) + relb_ref[...]
        srow = _col_to_row(score, NPC)
        gt = srow > score
        eq = srow == score
        rank = jnp.sum((gt | (eq & lt_idx)).astype(_F32), axis=1,
                       keepdims=True)
        P = (iop == _col_to_row(rank, NPC).astype(jnp.int32)).astype(_F32)
        xpool = _mm(P, x_c * jnp.tanh(score))
        cmax = jnp.max(xpool, axis=0, keepdims=True)
        cmean = jnp.sum(xpool, axis=0, keepdims=True) * (1.0 / K)
        cats.append(jnp.concatenate([cmax, cmean], axis=1))
    cat_emb = jnp.concatenate(cats, axis=0)

    xp_out = _mm(cat_emb, cgW_ref[...])
    catv = _gat_dense(xp_out, cgas_ref[...], cgad_ref[...], cgb_ref[...],
                      osrc_ref[...], odstc_ref[...], odstr_ref[...],
                      NCAT, E_OUT)

    B5 = (lax.broadcasted_iota(jnp.int32, (N, NCAT), 0) // NPC
          == lax.broadcasted_iota(jnp.int32, (N, NCAT), 1)).astype(_F32)
    catvB = _mm(B5, catv)
    fus_in = jnp.concatenate([wav, catvB, inner], axis=1)
    fus = jnp.maximum(_mm(fus_in, fusW_ref[...]) + fusb_ref[...], 0.0)
    reg_ref[...] = _mm(fus, regW_ref[...]) + regb_ref[...]
    cls_ref[...] = jax.nn.sigmoid(_mm(fus, clsW_ref[...]) + clsb_ref[...])


def kernel(w0, w1, w2, w3, inner_edge, inner20_edge, outer_edge, enc_W_ih,
           enc_W_hh, enc_b_ih, enc_b_hh, enc_att_W, enc_att_b, wg_W_ih,
           wg_W_hh, wg_b_ih, wg_b_hh, wa_W, wa_b, ig_W, ig_asrc, ig_adst,
           ig_b, sag_rel_W, sag_rel_b, sag_root_W, cg_W, cg_asrc, cg_adst,
           cg_b, fus_W, fus_b, reg_W, reg_b, cls_W, cls_b):
    wt = jnp.transpose(jnp.stack([w0, w1, w2, w3]), (0, 2, 1, 3))
    wt = jnp.pad(wt, ((0, 0), (0, 0), (0, 4), (0, 0)))
    wt = jnp.reshape(wt, (NWEEK, T * 104, D))
    loops = jnp.arange(N, dtype=jnp.int32)
    isrc = jnp.concatenate([inner_edge[0].astype(jnp.int32), loops])
    idst = jnp.concatenate([inner_edge[1].astype(jnp.int32), loops])
    oloops = jnp.arange(NCAT, dtype=jnp.int32)
    osrc = jnp.concatenate([outer_edge[0].astype(jnp.int32), oloops])
    odst = jnp.concatenate([outer_edge[1].astype(jnp.int32), oloops])
    s20 = inner20_edge[0].astype(jnp.int32)
    d20 = inner20_edge[1].astype(jnp.int32)

    args = (
        wt,
        jnp.transpose(enc_W_ih, (0, 2, 1)),
        jnp.transpose(enc_W_hh, (0, 2, 1)),
        enc_b_ih, enc_b_hh,
        enc_att_W,
        enc_att_b[:, :, None],
        wg_W_ih.T, wg_W_hh.T,
        wg_b_ih[None, :], wg_b_hh[None, :],
        wa_W, wa_b,
        ig_W.T, ig_asrc[None, :], ig_adst[None, :], ig_b[None, :],
        isrc[:, None], idst[:, None], idst[None, :],
        s20[:, None], d20[None, :],
        sag_rel_W.T, sag_rel_b[None, :], sag_root_W.T,
        cg_W.T, cg_asrc[None, :], cg_adst[None, :], cg_b[None, :],
        osrc[:, None], odst[:, None], odst[None, :],
        fus_W.T, fus_b[None, :], reg_W.T, reg_b[None, :],
        cls_W.T, cls_b[None, :],
    )
    vmem = pl.BlockSpec(memory_space=pltpu.VMEM)
    smem = pl.BlockSpec(memory_space=pltpu.SMEM)
    in_specs = [vmem] * len(args)
    in_specs[11] = smem
    in_specs[12] = smem

    reg, cls = pl.pallas_call(
        _forward_kernel,
        out_shape=[
            jax.ShapeDtypeStruct((N, 1), _F32),
            jax.ShapeDtypeStruct((N, 1), _F32),
        ],
        in_specs=in_specs,
        out_specs=[vmem, vmem],
        scratch_shapes=[
            pltpu.VMEM((NWEEK, T, N, H), _F32),
            pltpu.VMEM((NWEEK * 104, 3 * H), _F32),
            pltpu.VMEM((NWEEK * 104, H), _F32),
            pltpu.VMEM((NWEEK, T * 104, 3 * H), _F32),
        ],
    )(*args)
    return jnp.reshape(reg, (-1,)), jnp.reshape(cls, (-1,))


# speedup vs baseline: 1.0041x; 1.0041x over previous
import jax
import jax.numpy as jnp
from jax import lax
from jax.experimental import pallas as pl
from jax.experimental.pallas import tpu as pltpu

H = 128
T = 50
D = 16
NWEEK = 4
N = 100
NCAT = 5
NPC = 20
K = 10
E_IN = 800 + N
E20 = 100
E_OUT = 20 + NCAT

_F32 = jnp.float32


def _gru_cell(h, gi, gh):
    i_r = gi[:, 0:H]
    i_z = gi[:, H:2 * H]
    i_n = gi[:, 2 * H:3 * H]
    h_r = gh[:, 0:H]
    h_z = gh[:, H:2 * H]
    h_n = gh[:, 2 * H:3 * H]
    r = jax.nn.sigmoid(i_r + h_r)
    z = jax.nn.sigmoid(i_z + h_z)
    n = jnp.tanh(i_n + r * h_n)
    return (1.0 - z) * n + z * h


def _mm(a, b):
    return jnp.dot(a, b, preferred_element_type=_F32)


def _col_to_row(v, n):
    eye = lax.broadcasted_iota(jnp.int32, (n, n), 0) == lax.broadcasted_iota(
        jnp.int32, (n, n), 1)
    return jnp.sum(jnp.where(eye, jnp.broadcast_to(v, (n, n)), 0.0), axis=0,
                   keepdims=True)


def _gat_dense(xp, a_src_row, a_dst_row, bias_row, src_col, dst_col, dst_row,
               n, e):
    es = jnp.sum(xp * a_src_row, axis=1, keepdims=True)
    ed = jnp.sum(xp * a_dst_row, axis=1, keepdims=True)
    ids_e = lax.broadcasted_iota(jnp.int32, (e, n), 1)
    S = (ids_e == src_col).astype(_F32)
    Dm = (ids_e == dst_col).astype(_F32)
    DmTmask = lax.broadcasted_iota(jnp.int32, (n, e), 0) == dst_row
    DmT = DmTmask.astype(_F32)
    alpha = _mm(S, es) + _mm(Dm, ed)
    alpha = jnp.where(alpha >= 0.0, alpha, 0.2 * alpha)
    alpha_b = jnp.broadcast_to(jnp.reshape(alpha, (1, e)), (n, e))
    amax = jnp.max(jnp.where(DmTmask, alpha_b, -1e30), axis=1, keepdims=True)
    ex = jnp.exp(alpha - _mm(Dm, amax))
    den = _mm(DmT, ex)
    coef = ex / (_mm(Dm, den) + 1e-16)
    gathered = _mm(S, xp)
    out = _mm(DmT, coef * gathered)
    return out + bias_row


def _forward_kernel(
    wt_ref, eWih_ref, eWhh_ref, ebih_ref, ebhh_ref, attW_ref, attb_ref,
    gWih_ref, gWhh_ref, gbih_ref, gbhh_ref, waW_ref, wab_ref,
    igW_ref, igas_ref, igad_ref, igb_ref,
    isrc_ref, idstc_ref, idstr_ref,
    s20_ref, d20r_ref, relW_ref, relb_ref, rootW_ref,
    cgW_ref, cgas_ref, cgad_ref, cgb_ref,
    osrc_ref, odstc_ref, odstr_ref,
    fusW_ref, fusb_ref, regW_ref, regb_ref, clsW_ref, clsb_ref,
    reg_ref, cls_ref,
    hs_ref, ws_ref, ys_ref,
):
    WihTs = [eWih_ref[i] for i in range(NWEEK)]
    WhhTs = [eWhh_ref[i] for i in range(NWEEK)]
    bihs = [ebih_ref[i] for i in range(NWEEK)]
    bhhs = [ebhh_ref[i] for i in range(NWEEK)]

    def step(t, hcarry):
        off = pl.multiple_of(t * 104, 8)
        new = []
        for w in range(NWEEK):
            x_t = wt_ref[w, pl.ds(off, 104)][0:N]
            gi = _mm(x_t, WihTs[w]) + bihs[w]
            gh = _mm(hcarry[w], WhhTs[w]) + bhhs[w]
            hn = _gru_cell(hcarry[w], gi, gh)
            hs_ref[w, pl.ds(t, 1)] = jnp.reshape(hn, (1, N, H))
            new.append(hn)
        return tuple(new)

    lax.fori_loop(0, T, step,
                  tuple(jnp.zeros((N, H), _F32) for _ in range(NWEEK)))

    gWihT = gWih_ref[...]
    gWhhT = gWhh_ref[...]
    gbih = gbih_ref[...]
    gbhh = gbhh_ref[...]
    for i in range(NWEEK):
        X3 = hs_ref[i]
        X = jnp.reshape(X3, (T, N * H))
        aw = _mm(attW_ref[i], X) + attb_ref[i]
        m = jnp.max(aw, axis=0, keepdims=True)
        p = jnp.exp(aw - m)
        p = p / jnp.sum(p, axis=0, keepdims=True)
        p3 = jnp.reshape(p, (T, N, H))
        av = jnp.sum(p3 * X3, axis=0)
        ws_ref[i * 104:i * 104 + N, :] = _mm(av, gWihT) + gbih

    def step2(t, h):
        gi = jnp.concatenate(
            [ws_ref[pl.ds(w * 104 + t, 1), :] for w in range(NWEEK)], axis=0)
        gh = _mm(h, gWhhT) + gbhh
        hn = _gru_cell(h, gi, gh)
        for w in range(NWEEK):
            ys_ref[pl.ds(w * 104 + t, 1), :] = hn[w:w + 1, :]
        return hn

    lax.fori_loop(0, N, step2, jnp.zeros((NWEEK, H), _F32))

    ys_w = [ys_ref[w * 104:w * 104 + N, :] for w in range(NWEEK)]
    aw_s = []
    for s in range(NWEEK):
        acc = jnp.zeros((N, H), _F32)
        for t in range(NWEEK):
            acc = acc + waW_ref[s, t] * ys_w[t]
        aw_s.append(acc + wab_ref[s])
    mx = jnp.maximum(jnp.maximum(aw_s[0], aw_s[1]),
                     jnp.maximum(aw_s[2], aw_s[3]))
    e_s = [jnp.exp(a - mx) for a in aw_s]
    den = e_s[0] + e_s[1] + e_s[2] + e_s[3]
    wav = jnp.zeros((N, H), _F32)
    for s in range(NWEEK):
        wav = wav + (e_s[s] / den) * ys_w[s]

    xp_in = _mm(wav, igW_ref[...])
    inner = _gat_dense(xp_in, igas_ref[...], igad_ref[...], igb_ref[...],
                       isrc_ref[...], idstc_ref[...], idstr_ref[...],
                       N, E_IN)

    ids20 = lax.broadcasted_iota(jnp.int32, (E20, NPC), 1)
    S20 = (ids20 == s20_ref[...]).astype(_F32)
    D20T = (lax.broadcasted_iota(jnp.int32, (NPC, E20), 0)
            == d20r_ref[...]).astype(_F32)
    M20 = _mm(D20T, S20)
    relWT = relW_ref[...]
    rootWT = rootW_ref[...]
    io_i = lax.broadcasted_iota(jnp.int32, (NPC, NPC), 0)
    io_j = lax.broadcasted_iota(jnp.int32, (NPC, NPC), 1)
    lt_idx = io_j < io_i
    iop = lax.broadcasted_iota(jnp.int32, (K, NPC), 0)
    cats = []
    for c in range(NCAT):
        x_c = wav[c * NPC:(c + 1) * NPC, :]
        aggr = _mm(M20, x_c)
        score = _mm(aggr, relWT) + _mm(x_c, rootWT) + relb_ref[...]
        srow = _col_to_row(score, NPC)
        gt = srow > score
        eq = srow == score
        rank = jnp.sum((gt | (eq & lt_idx)).astype(_F32), axis=1,
                       keepdims=True)
        P = (iop == _col_to_row(rank, NPC).astype(jnp.int32)).astype(_F32)
        xpool = _mm(P, x_c * jnp.tanh(score))
        cmax = jnp.max(xpool, axis=0, keepdims=True)
        cmean = jnp.sum(xpool, axis=0, keepdims=True) * (1.0 / K)
        cats.append(jnp.concatenate([cmax, cmean], axis=1))
    cat_emb = jnp.concatenate(cats, axis=0)

    xp_out = _mm(cat_emb, cgW_ref[...])
    catv = _gat_dense(xp_out, cgas_ref[...], cgad_ref[...], cgb_ref[...],
                      osrc_ref[...], odstc_ref[...], odstr_ref[...],
                      NCAT, E_OUT)

    B5 = (lax.broadcasted_iota(jnp.int32, (N, NCAT), 0) // NPC
          == lax.broadcasted_iota(jnp.int32, (N, NCAT), 1)).astype(_F32)
    catvB = _mm(B5, catv)
    fus_in = jnp.concatenate([wav, catvB, inner], axis=1)
    fus = jnp.maximum(_mm(fus_in, fusW_ref[...]) + fusb_ref[...], 0.0)
    reg_ref[...] = _mm(fus, regW_ref[...]) + regb_ref[...]
    cls_ref[...] = jax.nn.sigmoid(_mm(fus, clsW_ref[...]) + clsb_ref[...])


def kernel(w0, w1, w2, w3, inner_edge, inner20_edge, outer_edge, enc_W_ih,
           enc_W_hh, enc_b_ih, enc_b_hh, enc_att_W, enc_att_b, wg_W_ih,
           wg_W_hh, wg_b_ih, wg_b_hh, wa_W, wa_b, ig_W, ig_asrc, ig_adst,
           ig_b, sag_rel_W, sag_rel_b, sag_root_W, cg_W, cg_asrc, cg_adst,
           cg_b, fus_W, fus_b, reg_W, reg_b, cls_W, cls_b):
    wt = jnp.transpose(jnp.stack([w0, w1, w2, w3]), (0, 2, 1, 3))
    wt = jnp.pad(wt, ((0, 0), (0, 0), (0, 4), (0, 0)))
    wt = jnp.reshape(wt, (NWEEK, T * 104, D))
    loops = jnp.arange(N, dtype=jnp.int32)
    isrc = jnp.concatenate([inner_edge[0].astype(jnp.int32), loops])
    idst = jnp.concatenate([inner_edge[1].astype(jnp.int32), loops])
    oloops = jnp.arange(NCAT, dtype=jnp.int32)
    osrc = jnp.concatenate([outer_edge[0].astype(jnp.int32), oloops])
    odst = jnp.concatenate([outer_edge[1].astype(jnp.int32), oloops])
    s20 = inner20_edge[0].astype(jnp.int32)
    d20 = inner20_edge[1].astype(jnp.int32)

    args = (
        wt,
        jnp.transpose(enc_W_ih, (0, 2, 1)),
        jnp.transpose(enc_W_hh, (0, 2, 1)),
        enc_b_ih, enc_b_hh,
        enc_att_W,
        enc_att_b[:, :, None],
        wg_W_ih.T, wg_W_hh.T,
        wg_b_ih[None, :], wg_b_hh[None, :],
        wa_W, wa_b,
        ig_W.T, ig_asrc[None, :], ig_adst[None, :], ig_b[None, :],
        isrc[:, None], idst[:, None], idst[None, :],
        s20[:, None], d20[None, :],
        sag_rel_W.T, sag_rel_b[None, :], sag_root_W.T,
        cg_W.T, cg_asrc[None, :], cg_adst[None, :], cg_b[None, :],
        osrc[:, None], odst[:, None], odst[None, :],
        fus_W.T, fus_b[None, :], reg_W.T, reg_b[None, :],
        cls_W.T, cls_b[None, :],
    )
    vmem = pl.BlockSpec(memory_space=pltpu.VMEM)
    smem = pl.BlockSpec(memory_space=pltpu.SMEM)
    in_specs = [vmem] * len(args)
    in_specs[11] = smem
    in_specs[12] = smem

    reg, cls = pl.pallas_call(
        _forward_kernel,
        out_shape=[
            jax.ShapeDtypeStruct((N, 1), _F32),
            jax.ShapeDtypeStruct((N, 1), _F32),
        ],
        in_specs=in_specs,
        out_specs=[vmem, vmem],
        scratch_shapes=[
            pltpu.VMEM((NWEEK, T, N, H), _F32),
            pltpu.VMEM((NWEEK * 104, 3 * H), _F32),
            pltpu.VMEM((NWEEK * 104, H), _F32),
        ],
    )(*args)
    return jnp.reshape(reg, (-1,)), jnp.reshape(cls, (-1,))


# speedup vs baseline: 1.1782x; 1.1735x over previous
import jax
import jax.numpy as jnp
from jax import lax
from jax.experimental import pallas as pl
from jax.experimental.pallas import tpu as pltpu

H = 128
T = 50
D = 16
NWEEK = 4
N = 100
NCAT = 5
NPC = 20
K = 10
E_IN = 800 + N
E20 = 100
E_OUT = 20 + NCAT

_F32 = jnp.float32


def _gru_cell(h, gi, gh):
    i_r = gi[:, 0:H]
    i_z = gi[:, H:2 * H]
    i_n = gi[:, 2 * H:3 * H]
    h_r = gh[:, 0:H]
    h_z = gh[:, H:2 * H]
    h_n = gh[:, 2 * H:3 * H]
    r = jax.nn.sigmoid(i_r + h_r)
    z = jax.nn.sigmoid(i_z + h_z)
    n = jnp.tanh(i_n + r * h_n)
    return (1.0 - z) * n + z * h


def _mm(a, b):
    return jnp.dot(a, b, preferred_element_type=_F32)


def _col_to_row(v, n):
    eye = lax.broadcasted_iota(jnp.int32, (n, n), 0) == lax.broadcasted_iota(
        jnp.int32, (n, n), 1)
    return jnp.sum(jnp.where(eye, jnp.broadcast_to(v, (n, n)), 0.0), axis=0,
                   keepdims=True)


def _gat_dense(xp, a_src_row, a_dst_row, bias_row, src_col, dst_col, dst_row,
               n, e):
    es = jnp.sum(xp * a_src_row, axis=1, keepdims=True)
    ed = jnp.sum(xp * a_dst_row, axis=1, keepdims=True)
    ids_e = lax.broadcasted_iota(jnp.int32, (e, n), 1)
    S = (ids_e == src_col).astype(_F32)
    Dm = (ids_e == dst_col).astype(_F32)
    DmTmask = lax.broadcasted_iota(jnp.int32, (n, e), 0) == dst_row
    DmT = DmTmask.astype(_F32)
    alpha = _mm(S, es) + _mm(Dm, ed)
    alpha = jnp.where(alpha >= 0.0, alpha, 0.2 * alpha)
    alpha_b = jnp.broadcast_to(jnp.reshape(alpha, (1, e)), (n, e))
    amax = jnp.max(jnp.where(DmTmask, alpha_b, -1e30), axis=1, keepdims=True)
    ex = jnp.exp(alpha - _mm(Dm, amax))
    den = _mm(DmT, ex)
    coef = ex / (_mm(Dm, den) + 1e-16)
    gathered = _mm(S, xp)
    out = _mm(DmT, coef * gathered)
    return out + bias_row


def _forward_kernel(
    wt_ref, eWih_ref, eWhh_ref, ebih_ref, ebhh_ref, attW_ref, attb_ref,
    gWih_ref, gWhh_ref, gbih_ref, gbhh_ref, waW_ref, wab_ref,
    igW_ref, igas_ref, igad_ref, igb_ref,
    isrc_ref, idstc_ref, idstr_ref,
    s20_ref, d20r_ref, relW_ref, relb_ref, rootW_ref,
    cgW_ref, cgas_ref, cgad_ref, cgb_ref,
    osrc_ref, odstc_ref, odstr_ref,
    fusW_ref, fusb_ref, regW_ref, regb_ref, clsW_ref, clsb_ref,
    reg_ref, cls_ref,
    hs_ref, ws_ref, ys_ref,
):
    WihTs = [eWih_ref[i] for i in range(NWEEK)]
    WhhTs = [eWhh_ref[i] for i in range(NWEEK)]
    bihs = [ebih_ref[i] for i in range(NWEEK)]
    bhhs = [ebhh_ref[i] for i in range(NWEEK)]

    def step(t, hcarry):
        new = []
        for w in range(NWEEK):
            x_t = jnp.reshape(wt_ref[w, pl.ds(t, 1)], (N, D))
            gi = _mm(x_t, WihTs[w]) + bihs[w]
            gh = _mm(hcarry[w], WhhTs[w]) + bhhs[w]
            hn = _gru_cell(hcarry[w], gi, gh)
            hs_ref[w, pl.ds(t, 1)] = jnp.reshape(hn, (1, N, H))
            new.append(hn)
        return tuple(new)

    lax.fori_loop(0, T, step,
                  tuple(jnp.zeros((N, H), _F32) for _ in range(NWEEK)),
                  unroll=2)

    for i in range(NWEEK):
        X = jnp.reshape(hs_ref[i], (T, N * H))
        aw = _mm(attW_ref[i], X) + attb_ref[i]
        m = jnp.max(aw, axis=0, keepdims=True)
        p = jnp.exp(aw - m)
        p = p / jnp.sum(p, axis=0, keepdims=True)
        av = jnp.sum(p * X, axis=0, keepdims=True)
        ws_ref[i:i + 1, :] = av

    gWihT = gWih_ref[...]
    gWhhT = gWhh_ref[...]
    gbih = gbih_ref[...]
    gbhh = gbhh_ref[...]

    def step2(t, h):
        off = pl.multiple_of(t * H, H)
        x_t = ws_ref[:, pl.ds(off, H)]
        gi = _mm(x_t, gWihT) + gbih
        gh = _mm(h, gWhhT) + gbhh
        hn = _gru_cell(h, gi, gh)
        for w in range(NWEEK):
            ys_ref[pl.ds(w * 104 + t, 1), :] = hn[w:w + 1, :]
        return hn

    lax.fori_loop(0, N, step2, jnp.zeros((NWEEK, H), _F32), unroll=2)

    ys_w = [ys_ref[w * 104:w * 104 + N, :] for w in range(NWEEK)]
    aw_s = []
    for s in range(NWEEK):
        acc = jnp.zeros((N, H), _F32)
        for t in range(NWEEK):
            acc = acc + waW_ref[s, t] * ys_w[t]
        aw_s.append(acc + wab_ref[s])
    mx = jnp.maximum(jnp.maximum(aw_s[0], aw_s[1]),
                     jnp.maximum(aw_s[2], aw_s[3]))
    e_s = [jnp.exp(a - mx) for a in aw_s]
    den = e_s[0] + e_s[1] + e_s[2] + e_s[3]
    wav = jnp.zeros((N, H), _F32)
    for s in range(NWEEK):
        wav = wav + (e_s[s] / den) * ys_w[s]

    xp_in = _mm(wav, igW_ref[...])
    inner = _gat_dense(xp_in, igas_ref[...], igad_ref[...], igb_ref[...],
                       isrc_ref[...], idstc_ref[...], idstr_ref[...],
                       N, E_IN)

    ids20 = lax.broadcasted_iota(jnp.int32, (E20, NPC), 1)
    S20 = (ids20 == s20_ref[...]).astype(_F32)
    D20T = (lax.broadcasted_iota(jnp.int32, (NPC, E20), 0)
            == d20r_ref[...]).astype(_F32)
    M20 = _mm(D20T, S20)
    relWT = relW_ref[...]
    rootWT = rootW_ref[...]
    io_i = lax.broadcasted_iota(jnp.int32, (NPC, NPC), 0)
    io_j = lax.broadcasted_iota(jnp.int32, (NPC, NPC), 1)
    lt_idx = io_j < io_i
    iop = lax.broadcasted_iota(jnp.int32, (K, NPC), 0)
    cats = []
    for c in range(NCAT):
        x_c = wav[c * NPC:(c + 1) * NPC, :]
        aggr = _mm(M20, x_c)
        score = _mm(aggr, relWT) + _mm(x_c, rootWT) + relb_ref[...]
        srow = _col_to_row(score, NPC)
        gt = srow > score
        eq = srow == score
        rank = jnp.sum((gt | (eq & lt_idx)).astype(_F32), axis=1,
                       keepdims=True)
        P = (iop == _col_to_row(rank, NPC).astype(jnp.int32)).astype(_F32)
        xpool = _mm(P, x_c * jnp.tanh(score))
        cmax = jnp.max(xpool, axis=0, keepdims=True)
        cmean = jnp.sum(xpool, axis=0, keepdims=True) * (1.0 / K)
        cats.append(jnp.concatenate([cmax, cmean], axis=1))
    cat_emb = jnp.concatenate(cats, axis=0)

    xp_out = _mm(cat_emb, cgW_ref[...])
    catv = _gat_dense(xp_out, cgas_ref[...], cgad_ref[...], cgb_ref[...],
                      osrc_ref[...], odstc_ref[...], odstr_ref[...],
                      NCAT, E_OUT)

    B5 = (lax.broadcasted_iota(jnp.int32, (N, NCAT), 0) // NPC
          == lax.broadcasted_iota(jnp.int32, (N, NCAT), 1)).astype(_F32)
    catvB = _mm(B5, catv)
    fus_in = jnp.concatenate([wav, catvB, inner], axis=1)
    fus = jnp.maximum(_mm(fus_in, fusW_ref[...]) + fusb_ref[...], 0.0)
    reg_ref[...] = _mm(fus, regW_ref[...]) + regb_ref[...]
    cls_ref[...] = jax.nn.sigmoid(_mm(fus, clsW_ref[...]) + clsb_ref[...])


def kernel(w0, w1, w2, w3, inner_edge, inner20_edge, outer_edge, enc_W_ih,
           enc_W_hh, enc_b_ih, enc_b_hh, enc_att_W, enc_att_b, wg_W_ih,
           wg_W_hh, wg_b_ih, wg_b_hh, wa_W, wa_b, ig_W, ig_asrc, ig_adst,
           ig_b, sag_rel_W, sag_rel_b, sag_root_W, cg_W, cg_asrc, cg_adst,
           cg_b, fus_W, fus_b, reg_W, reg_b, cls_W, cls_b):
    wt = jnp.transpose(jnp.stack([w0, w1, w2, w3]), (0, 2, 1, 3))
    loops = jnp.arange(N, dtype=jnp.int32)
    isrc = jnp.concatenate([inner_edge[0].astype(jnp.int32), loops])
    idst = jnp.concatenate([inner_edge[1].astype(jnp.int32), loops])
    oloops = jnp.arange(NCAT, dtype=jnp.int32)
    osrc = jnp.concatenate([outer_edge[0].astype(jnp.int32), oloops])
    odst = jnp.concatenate([outer_edge[1].astype(jnp.int32), oloops])
    s20 = inner20_edge[0].astype(jnp.int32)
    d20 = inner20_edge[1].astype(jnp.int32)

    args = (
        wt,
        jnp.transpose(enc_W_ih, (0, 2, 1)),
        jnp.transpose(enc_W_hh, (0, 2, 1)),
        enc_b_ih, enc_b_hh,
        enc_att_W,
        enc_att_b[:, :, None],
        wg_W_ih.T, wg_W_hh.T,
        wg_b_ih[None, :], wg_b_hh[None, :],
        wa_W, wa_b,
        ig_W.T, ig_asrc[None, :], ig_adst[None, :], ig_b[None, :],
        isrc[:, None], idst[:, None], idst[None, :],
        s20[:, None], d20[None, :],
        sag_rel_W.T, sag_rel_b[None, :], sag_root_W.T,
        cg_W.T, cg_asrc[None, :], cg_adst[None, :], cg_b[None, :],
        osrc[:, None], odst[:, None], odst[None, :],
        fus_W.T, fus_b[None, :], reg_W.T, reg_b[None, :],
        cls_W.T, cls_b[None, :],
    )
    vmem = pl.BlockSpec(memory_space=pltpu.VMEM)
    smem = pl.BlockSpec(memory_space=pltpu.SMEM)
    in_specs = [vmem] * len(args)
    in_specs[11] = smem
    in_specs[12] = smem

    reg, cls = pl.pallas_call(
        _forward_kernel,
        out_shape=[
            jax.ShapeDtypeStruct((N, 1), _F32),
            jax.ShapeDtypeStruct((N, 1), _F32),
        ],
        in_specs=in_specs,
        out_specs=[vmem, vmem],
        scratch_shapes=[
            pltpu.VMEM((NWEEK, T, N, H), _F32),
            pltpu.VMEM((NWEEK, N * H), _F32),
            pltpu.VMEM((NWEEK * 104, H), _F32),
        ],
    )(*args)
    return jnp.reshape(reg, (-1,)), jnp.reshape(cls, (-1,))


# speedup vs baseline: 1.2377x; 1.0505x over previous
import jax
import jax.numpy as jnp
from jax import lax
from jax.experimental import pallas as pl
from jax.experimental.pallas import tpu as pltpu

H = 128
T = 50
D = 16
NWEEK = 4
N = 100
NCAT = 5
NPC = 20
K = 10
E_IN = 800 + N
E20 = 100
E_OUT = 20 + NCAT

_F32 = jnp.float32


def _gru_cell(h, gi, gh):
    i_r = gi[:, 0:H]
    i_z = gi[:, H:2 * H]
    i_n = gi[:, 2 * H:3 * H]
    h_r = gh[:, 0:H]
    h_z = gh[:, H:2 * H]
    h_n = gh[:, 2 * H:3 * H]
    r = jax.nn.sigmoid(i_r + h_r)
    z = jax.nn.sigmoid(i_z + h_z)
    n = jnp.tanh(i_n + r * h_n)
    return (1.0 - z) * n + z * h


def _mm(a, b):
    return jnp.dot(a, b, preferred_element_type=_F32)


def _col_to_row(v, n):
    eye = lax.broadcasted_iota(jnp.int32, (n, n), 0) == lax.broadcasted_iota(
        jnp.int32, (n, n), 1)
    return jnp.sum(jnp.where(eye, jnp.broadcast_to(v, (n, n)), 0.0), axis=0,
                   keepdims=True)


def _gat_dense(xp, a_src_row, a_dst_row, bias_row, src_col, dst_col, dst_row,
               n, e):
    es = jnp.sum(xp * a_src_row, axis=1, keepdims=True)
    ed = jnp.sum(xp * a_dst_row, axis=1, keepdims=True)
    ids_e = lax.broadcasted_iota(jnp.int32, (e, n), 1)
    S = (ids_e == src_col).astype(_F32)
    Dm = (ids_e == dst_col).astype(_F32)
    DmTmask = lax.broadcasted_iota(jnp.int32, (n, e), 0) == dst_row
    DmT = DmTmask.astype(_F32)
    alpha = _mm(S, es) + _mm(Dm, ed)
    alpha = jnp.where(alpha >= 0.0, alpha, 0.2 * alpha)
    alpha_b = jnp.broadcast_to(jnp.reshape(alpha, (1, e)), (n, e))
    amax = jnp.max(jnp.where(DmTmask, alpha_b, -1e30), axis=1, keepdims=True)
    ex = jnp.exp(alpha - _mm(Dm, amax))
    den = _mm(DmT, ex)
    coef = ex / (_mm(Dm, den) + 1e-16)
    gathered = _mm(S, xp)
    out = _mm(DmT, coef * gathered)
    return out + bias_row


def _forward_kernel(
    wt_ref, eWih_ref, eWhh_ref, ebih_ref, ebhh_ref, attW_ref, attb_ref,
    gWih_ref, gWhh_ref, gbih_ref, gbhh_ref, waW_ref, wab_ref,
    igW_ref, igas_ref, igad_ref, igb_ref,
    isrc_ref, idstc_ref, idstr_ref,
    s20_ref, d20r_ref, relW_ref, relb_ref, rootW_ref,
    cgW_ref, cgas_ref, cgad_ref, cgb_ref,
    osrc_ref, odstc_ref, odstr_ref,
    fusW_ref, fusb_ref, regW_ref, regb_ref, clsW_ref, clsb_ref,
    reg_ref, cls_ref,
    hs_ref, ws_ref, ys_ref,
):
    WihTs = [eWih_ref[i] for i in range(NWEEK)]
    WhhTs = [eWhh_ref[i] for i in range(NWEEK)]
    bihs = [ebih_ref[i] for i in range(NWEEK)]
    bhhs = [ebhh_ref[i] for i in range(NWEEK)]

    def step(t, hcarry):
        new = []
        for w in range(NWEEK):
            x_t = jnp.reshape(wt_ref[w, pl.ds(t, 1)], (N, D))
            gi = _mm(x_t, WihTs[w]) + bihs[w]
            gh = _mm(hcarry[w], WhhTs[w]) + bhhs[w]
            hn = _gru_cell(hcarry[w], gi, gh)
            hs_ref[w, pl.ds(t, 1)] = jnp.reshape(hn, (1, N, H))
            new.append(hn)
        return tuple(new)

    lax.fori_loop(0, T, step,
                  tuple(jnp.zeros((N, H), _F32) for _ in range(NWEEK)),
                  unroll=5)

    for i in range(NWEEK):
        X = jnp.reshape(hs_ref[i], (T, N * H))
        aw = _mm(attW_ref[i], X) + attb_ref[i]
        m = jnp.max(aw, axis=0, keepdims=True)
        p = jnp.exp(aw - m)
        p = p / jnp.sum(p, axis=0, keepdims=True)
        av = jnp.sum(p * X, axis=0, keepdims=True)
        ws_ref[i:i + 1, :] = av

    gWihT = gWih_ref[...]
    gWhhT = gWhh_ref[...]
    gbih = gbih_ref[...]
    gbhh = gbhh_ref[...]

    def step2(t, h):
        off = pl.multiple_of(t * H, H)
        x_t = ws_ref[:, pl.ds(off, H)]
        gi = _mm(x_t, gWihT) + gbih
        gh = _mm(h, gWhhT) + gbhh
        hn = _gru_cell(h, gi, gh)
        for w in range(NWEEK):
            ys_ref[pl.ds(w * 104 + t, 1), :] = hn[w:w + 1, :]
        return hn

    lax.fori_loop(0, N, step2, jnp.zeros((NWEEK, H), _F32), unroll=4)

    ys_w = [ys_ref[w * 104:w * 104 + N, :] for w in range(NWEEK)]
    aw_s = []
    for s in range(NWEEK):
        acc = jnp.zeros((N, H), _F32)
        for t in range(NWEEK):
            acc = acc + waW_ref[s, t] * ys_w[t]
        aw_s.append(acc + wab_ref[s])
    mx = jnp.maximum(jnp.maximum(aw_s[0], aw_s[1]),
                     jnp.maximum(aw_s[2], aw_s[3]))
    e_s = [jnp.exp(a - mx) for a in aw_s]
    den = e_s[0] + e_s[1] + e_s[2] + e_s[3]
    wav = jnp.zeros((N, H), _F32)
    for s in range(NWEEK):
        wav = wav + (e_s[s] / den) * ys_w[s]

    xp_in = _mm(wav, igW_ref[...])
    inner = _gat_dense(xp_in, igas_ref[...], igad_ref[...], igb_ref[...],
                       isrc_ref[...], idstc_ref[...], idstr_ref[...],
                       N, E_IN)

    ids20 = lax.broadcasted_iota(jnp.int32, (E20, NPC), 1)
    S20 = (ids20 == s20_ref[...]).astype(_F32)
    D20T = (lax.broadcasted_iota(jnp.int32, (NPC, E20), 0)
            == d20r_ref[...]).astype(_F32)
    M20 = _mm(D20T, S20)
    relWT = relW_ref[...]
    rootWT = rootW_ref[...]
    io_i = lax.broadcasted_iota(jnp.int32, (NPC, NPC), 0)
    io_j = lax.broadcasted_iota(jnp.int32, (NPC, NPC), 1)
    lt_idx = io_j < io_i
    iop = lax.broadcasted_iota(jnp.int32, (K, NPC), 0)
    cats = []
    for c in range(NCAT):
        x_c = wav[c * NPC:(c + 1) * NPC, :]
        aggr = _mm(M20, x_c)
        score = _mm(aggr, relWT) + _mm(x_c, rootWT) + relb_ref[...]
        srow = _col_to_row(score, NPC)
        gt = srow > score
        eq = srow == score
        rank = jnp.sum((gt | (eq & lt_idx)).astype(_F32), axis=1,
                       keepdims=True)
        P = (iop == _col_to_row(rank, NPC).astype(jnp.int32)).astype(_F32)
        xpool = _mm(P, x_c * jnp.tanh(score))
        cmax = jnp.max(xpool, axis=0, keepdims=True)
        cmean = jnp.sum(xpool, axis=0, keepdims=True) * (1.0 / K)
        cats.append(jnp.concatenate([cmax, cmean], axis=1))
    cat_emb = jnp.concatenate(cats, axis=0)

    xp_out = _mm(cat_emb, cgW_ref[...])
    catv = _gat_dense(xp_out, cgas_ref[...], cgad_ref[...], cgb_ref[...],
                      osrc_ref[...], odstc_ref[...], odstr_ref[...],
                      NCAT, E_OUT)

    B5 = (lax.broadcasted_iota(jnp.int32, (N, NCAT), 0) // NPC
          == lax.broadcasted_iota(jnp.int32, (N, NCAT), 1)).astype(_F32)
    catvB = _mm(B5, catv)
    fus_in = jnp.concatenate([wav, catvB, inner], axis=1)
    fus = jnp.maximum(_mm(fus_in, fusW_ref[...]) + fusb_ref[...], 0.0)
    reg_ref[...] = _mm(fus, regW_ref[...]) + regb_ref[...]
    cls_ref[...] = jax.nn.sigmoid(_mm(fus, clsW_ref[...]) + clsb_ref[...])


def kernel(w0, w1, w2, w3, inner_edge, inner20_edge, outer_edge, enc_W_ih,
           enc_W_hh, enc_b_ih, enc_b_hh, enc_att_W, enc_att_b, wg_W_ih,
           wg_W_hh, wg_b_ih, wg_b_hh, wa_W, wa_b, ig_W, ig_asrc, ig_adst,
           ig_b, sag_rel_W, sag_rel_b, sag_root_W, cg_W, cg_asrc, cg_adst,
           cg_b, fus_W, fus_b, reg_W, reg_b, cls_W, cls_b):
    wt = jnp.transpose(jnp.stack([w0, w1, w2, w3]), (0, 2, 1, 3))
    loops = jnp.arange(N, dtype=jnp.int32)
    isrc = jnp.concatenate([inner_edge[0].astype(jnp.int32), loops])
    idst = jnp.concatenate([inner_edge[1].astype(jnp.int32), loops])
    oloops = jnp.arange(NCAT, dtype=jnp.int32)
    osrc = jnp.concatenate([outer_edge[0].astype(jnp.int32), oloops])
    odst = jnp.concatenate([outer_edge[1].astype(jnp.int32), oloops])
    s20 = inner20_edge[0].astype(jnp.int32)
    d20 = inner20_edge[1].astype(jnp.int32)

    args = (
        wt,
        jnp.transpose(enc_W_ih, (0, 2, 1)),
        jnp.transpose(enc_W_hh, (0, 2, 1)),
        enc_b_ih, enc_b_hh,
        enc_att_W,
        enc_att_b[:, :, None],
        wg_W_ih.T, wg_W_hh.T,
        wg_b_ih[None, :], wg_b_hh[None, :],
        wa_W, wa_b,
        ig_W.T, ig_asrc[None, :], ig_adst[None, :], ig_b[None, :],
        isrc[:, None], idst[:, None], idst[None, :],
        s20[:, None], d20[None, :],
        sag_rel_W.T, sag_rel_b[None, :], sag_root_W.T,
        cg_W.T, cg_asrc[None, :], cg_adst[None, :], cg_b[None, :],
        osrc[:, None], odst[:, None], odst[None, :],
        fus_W.T, fus_b[None, :], reg_W.T, reg_b[None, :],
        cls_W.T, cls_b[None, :],
    )
    vmem = pl.BlockSpec(memory_space=pltpu.VMEM)
    smem = pl.BlockSpec(memory_space=pltpu.SMEM)
    in_specs = [vmem] * len(args)
    in_specs[11] = smem
    in_specs[12] = smem

    reg, cls = pl.pallas_call(
        _forward_kernel,
        out_shape=[
            jax.ShapeDtypeStruct((N, 1), _F32),
            jax.ShapeDtypeStruct((N, 1), _F32),
        ],
        in_specs=in_specs,
        out_specs=[vmem, vmem],
        scratch_shapes=[
            pltpu.VMEM((NWEEK, T, N, H), _F32),
            pltpu.VMEM((NWEEK, N * H), _F32),
            pltpu.VMEM((NWEEK * 104, H), _F32),
        ],
    )(*args)
    return jnp.reshape(reg, (-1,)), jnp.reshape(cls, (-1,))
